# Initial kernel scaffold; baseline (speedup 1.0000x reference)
import jax
import jax.numpy as jnp
from jax.experimental import pallas as pl


def kernel(articles, moments, user_id, age, gender, W_feat, b_feat, W_mom, b_mom, UserEmb, AgeEmb, GenderEmb, W_out, b_out):
    raise NotImplementedError("write your pallas kernel here")


# speedup vs baseline: 2.4432x; 2.4432x over previous
import functools
import jax
import jax.numpy as jnp
from jax import lax
from jax.experimental import pallas as pl
from jax.experimental.pallas import tpu as pltpu
from jax.experimental.pallas import tpu_sc as plsc

B = 16384
E = 64
DOUT = 256
NC = 2
NS = 16
NW = NC * NS
BPW = B // NW
CHUNK = 128
NCHUNK = BPW // CHUNK


def _sc_gather_body(table_hbm, idx_hbm, out_hbm, idx_v, rows_v, sem):
    wid = lax.axis_index("s") * NC + lax.axis_index("c")
    base = wid * BPW
    pltpu.sync_copy(idx_hbm.at[wid], idx_v)
    copies = [
        pltpu.async_copy(
            table_hbm.at[idx_v.at[j]],
            rows_v.at[pl.ds(j * CHUNK, CHUNK)],
            sem,
        )
        for j in range(NCHUNK)
    ]
    for c in copies:
        c.wait()
    pltpu.sync_copy(rows_v, out_hbm.at[pl.ds(base, BPW)])


@functools.partial(jax.jit, static_argnames=())
def _sc_gather(table, idx3):
    mesh = plsc.VectorSubcoreMesh(
        core_axis_name="c", subcore_axis_name="s",
        num_cores=NC, num_subcores=NS,
    )
    return pl.kernel(
        _sc_gather_body,
        out_type=jax.ShapeDtypeStruct((B, E), jnp.float32),
        mesh=mesh,
        scratch_types=[
            pltpu.VMEM((NCHUNK, CHUNK), jnp.int32),
            pltpu.VMEM((BPW, E), jnp.float32),
            pltpu.SemaphoreType.DMA,
        ],
        compiler_params=pltpu.CompilerParams(use_tc_tiling_on_sc=False),
    )(table, idx3)


BS = 2048


def _tc_dense_body(uemb_ref, ag_ref, small_ref, w1_ref, w23_ref, bias_ref, out_ref):
    cmb = jnp.dot(small_ref[...], w23_ref[...],
                  preferred_element_type=jnp.float32)
    ag = ag_ref[...]
    iota8 = lax.broadcasted_iota(jnp.int32, (BS, 8), 1)
    mask = jnp.logical_or(iota8 == ag[:, 0:1], iota8 == ag[:, 1:2] + 6)
    onehot = mask.astype(jnp.float32)
    out_ref[...] = (
        jnp.dot(uemb_ref[...], w1_ref[...], preferred_element_type=jnp.float32)
        + jnp.dot(onehot, cmb, preferred_element_type=jnp.float32)
        + bias_ref[...]
    )


@jax.jit
def _tc_dense(uemb, ag, small, w1, w23, bias):
    grid = (B // BS,)
    return pl.pallas_call(
        _tc_dense_body,
        grid=grid,
        in_specs=[
            pl.BlockSpec((BS, E), lambda i: (i, 0)),
            pl.BlockSpec((BS, 2), lambda i: (i, 0)),
            pl.BlockSpec((8, 2 * E), lambda i: (0, 0)),
            pl.BlockSpec((E, DOUT), lambda i: (0, 0)),
            pl.BlockSpec((2 * E, DOUT), lambda i: (0, 0)),
            pl.BlockSpec((1, DOUT), lambda i: (0, 0)),
        ],
        out_specs=pl.BlockSpec((BS, DOUT), lambda i: (i, 0)),
        out_shape=jax.ShapeDtypeStruct((B, DOUT), jnp.float32),
    )(uemb, ag, small, w1, w23, bias)


def kernel(articles, moments, user_id, age, gender, W_feat, b_feat, W_mom,
           b_mom, UserEmb, AgeEmb, GenderEmb, W_out, b_out):
    idx3 = user_id.astype(jnp.int32).reshape(NW, NCHUNK, CHUNK)
    uemb = _sc_gather(UserEmb, idx3)

    ag = jnp.stack([age.astype(jnp.int32), gender.astype(jnp.int32)], axis=1)
    small = jnp.zeros((8, 2 * E), jnp.float32)
    small = small.at[0:6, 0:E].set(AgeEmb)
    small = small.at[6:8, E:2 * E].set(GenderEmb)
    w1 = W_out[0:E]
    w23 = W_out[E:3 * E]
    bias = b_out.reshape(1, DOUT)
    return _tc_dense(uemb, ag, small, w1, w23, bias)
